# Initial kernel scaffold; baseline (speedup 1.0000x reference)
import jax
import jax.numpy as jnp
from jax.experimental import pallas as pl


def kernel(x, router_w, w1, w_gate, w2):
    raise NotImplementedError("write your pallas kernel here")


# speedup vs baseline: 1.1473x; 1.1473x over previous
import jax
import jax.numpy as jnp
from jax.experimental import pallas as pl
from jax.experimental.pallas import tpu as pltpu


def _router_kernel(x_ref, rw_ref, comb_ref):
    x = x_ref[...]
    rw = rw_ref[...]
    logits = jax.lax.dot_general(
        x, rw, (((1,), (1,)), ((), ())), preferred_element_type=jnp.float32)
    t, e = logits.shape
    col = jax.lax.broadcasted_iota(jnp.int32, (t, e), 1)
    m1 = jnp.max(logits, axis=1, keepdims=True)
    i1 = jnp.min(jnp.where(logits == m1, col, e), axis=1, keepdims=True)
    masked = jnp.where(col == i1, -jnp.inf, logits)
    m2 = jnp.max(masked, axis=1, keepdims=True)
    i2 = jnp.min(jnp.where(masked == m2, col, e), axis=1, keepdims=True)
    b = jnp.exp(m2 - m1)
    w_hi = 1.0 / (1.0 + b)
    w_lo = b / (1.0 + b)
    comb_ref[...] = (jnp.where(col == i1, w_hi, 0.0)
                     + jnp.where(col == i2, w_lo, 0.0))


def _expert_kernel(sched_ref, x_ref, w1_ref, wg_ref, w2_ref, cw_ref, out_ref):
    del sched_ref
    i = pl.program_id(0)

    @pl.when(i == 0)
    def _():
        out_ref[...] = jnp.zeros_like(out_ref)

    x = x_ref[...]
    h1 = jnp.dot(x, w1_ref[0], preferred_element_type=jnp.float32)
    g = jnp.dot(x, wg_ref[0], preferred_element_type=jnp.float32)
    act = h1 * (g * jax.nn.sigmoid(g))
    oe = jnp.dot(act, w2_ref[0], preferred_element_type=jnp.float32)
    out_ref[...] += cw_ref[0] * oe


def kernel(x, router_w, w1, w_gate, w2):
    orig_shape = x.shape
    d = x.shape[-1]
    xf = x.reshape(-1, d)
    t = xf.shape[0]
    e = router_w.shape[0]
    h = w1.shape[2]

    comb = pl.pallas_call(
        _router_kernel,
        out_shape=jax.ShapeDtypeStruct((t, e), jnp.float32),
    )(xf, router_w)

    sel = jnp.any(comb > 0.0, axis=0)
    order = jnp.argsort(~sel, stable=True).astype(jnp.int32)
    n = jnp.sum(sel).astype(jnp.int32)
    last = order[jnp.maximum(n - 1, 0)]
    steps = jnp.arange(e, dtype=jnp.int32)
    sched = jnp.where(steps < n, order, last)
    cw = jnp.where(steps[:, None] < n, comb.T[sched], 0.0)
    cw3 = cw[:, :, None]

    grid_spec = pltpu.PrefetchScalarGridSpec(
        num_scalar_prefetch=1,
        grid=(e,),
        in_specs=[
            pl.BlockSpec((t, d), lambda i, s: (0, 0)),
            pl.BlockSpec((1, d, h), lambda i, s: (s[i], 0, 0)),
            pl.BlockSpec((1, d, h), lambda i, s: (s[i], 0, 0)),
            pl.BlockSpec((1, h, d), lambda i, s: (s[i], 0, 0)),
            pl.BlockSpec((1, t, 1), lambda i, s: (i, 0, 0)),
        ],
        out_specs=pl.BlockSpec((t, d), lambda i, s: (0, 0)),
    )
    out = pl.pallas_call(
        _expert_kernel,
        grid_spec=grid_spec,
        out_shape=jax.ShapeDtypeStruct((t, d), jnp.float32),
    )(sched, xf, w1, w_gate, w2, cw3)
    return out.reshape(orig_shape)


# speedup vs baseline: 1.2073x; 1.0523x over previous
import jax
import jax.numpy as jnp
from jax.experimental import pallas as pl
from jax.experimental.pallas import tpu as pltpu


def _router_kernel(x_ref, rw_ref, comb_ref):
    x = x_ref[...]
    rw = rw_ref[...]
    logits = jax.lax.dot_general(
        x, rw, (((1,), (1,)), ((), ())), preferred_element_type=jnp.float32)
    t, e = logits.shape
    col = jax.lax.broadcasted_iota(jnp.int32, (t, e), 1)
    m1 = jnp.max(logits, axis=1, keepdims=True)
    i1 = jnp.min(jnp.where(logits == m1, col, e), axis=1, keepdims=True)
    masked = jnp.where(col == i1, -jnp.inf, logits)
    m2 = jnp.max(masked, axis=1, keepdims=True)
    i2 = jnp.min(jnp.where(masked == m2, col, e), axis=1, keepdims=True)
    b = jnp.exp(m2 - m1)
    w_hi = 1.0 / (1.0 + b)
    w_lo = b / (1.0 + b)
    comb_ref[...] = (jnp.where(col == i1, w_hi, 0.0)
                     + jnp.where(col == i2, w_lo, 0.0))


def _expert_kernel(sched_ref, x_ref, w1_ref, wg_ref, w2_ref, cw_ref, out_ref):
    i = pl.program_id(0)
    n = sched_ref[sched_ref.shape[0] - 1]

    @pl.when(i == 0)
    def _():
        out_ref[...] = jnp.zeros_like(out_ref)

    @pl.when(i < n)
    def _():
        x = x_ref[...]
        h1 = jnp.dot(x, w1_ref[0], preferred_element_type=jnp.float32)
        g = jnp.dot(x, wg_ref[0], preferred_element_type=jnp.float32)
        act = h1 * (g * jax.nn.sigmoid(g))
        oe = jnp.dot(act, w2_ref[0], preferred_element_type=jnp.float32)
        out_ref[...] += cw_ref[i] * oe


def kernel(x, router_w, w1, w_gate, w2):
    orig_shape = x.shape
    d = x.shape[-1]
    xf = x.reshape(-1, d)
    t = xf.shape[0]
    e = router_w.shape[0]
    h = w1.shape[2]

    comb = pl.pallas_call(
        _router_kernel,
        out_shape=jax.ShapeDtypeStruct((t, e), jnp.float32),
    )(xf, router_w)

    sel = jnp.any(comb > 0.0, axis=0)
    order = jnp.argsort(~sel, stable=True).astype(jnp.int32)
    n = jnp.sum(sel).astype(jnp.int32)
    last = order[jnp.maximum(n - 1, 0)]
    steps = jnp.arange(e, dtype=jnp.int32)
    sched = jnp.where(steps < n, order, last)
    sched = jnp.concatenate([sched, n[None]])
    cw3 = comb.T[sched[:-1]][:, :, None]

    grid_spec = pltpu.PrefetchScalarGridSpec(
        num_scalar_prefetch=1,
        grid=(e,),
        in_specs=[
            pl.BlockSpec((t, d), lambda i, s: (0, 0)),
            pl.BlockSpec((1, d, h), lambda i, s: (s[i], 0, 0)),
            pl.BlockSpec((1, d, h), lambda i, s: (s[i], 0, 0)),
            pl.BlockSpec((1, h, d), lambda i, s: (s[i], 0, 0)),
            pl.BlockSpec((e, t, 1), lambda i, s: (0, 0, 0)),
        ],
        out_specs=pl.BlockSpec((t, d), lambda i, s: (0, 0)),
    )
    out = pl.pallas_call(
        _expert_kernel,
        grid_spec=grid_spec,
        out_shape=jax.ShapeDtypeStruct((t, d), jnp.float32),
    )(sched, xf, w1, w_gate, w2, cw3)
    return out.reshape(orig_shape)


# speedup vs baseline: 1.2490x; 1.0345x over previous
import jax
import jax.numpy as jnp
from jax.experimental import pallas as pl
from jax.experimental.pallas import tpu as pltpu


def _router_kernel(x_ref, rw_ref, sched_ref, n_ref, cw_ref):
    x = x_ref[...]
    rw = rw_ref[...]
    logits = jax.lax.dot_general(
        rw, x, (((1,), (1,)), ((), ())), preferred_element_type=jnp.float32)
    e, t = logits.shape
    row = jax.lax.broadcasted_iota(jnp.int32, (e, t), 0)
    m1 = jnp.max(logits, axis=0, keepdims=True)
    i1 = jnp.min(jnp.where(logits == m1, row, e), axis=0, keepdims=True)
    masked = jnp.where(row == i1, -jnp.inf, logits)
    m2 = jnp.max(masked, axis=0, keepdims=True)
    i2 = jnp.min(jnp.where(masked == m2, row, e), axis=0, keepdims=True)
    b = jnp.exp(m2 - m1)
    w_hi = 1.0 / (1.0 + b)
    w_lo = b / (1.0 + b)
    comb = (jnp.where(row == i1, w_hi, 0.0)
            + jnp.where(row == i2, w_lo, 0.0))

    sel = jnp.max(comb, axis=1, keepdims=True) > 0.0
    ee_r = jax.lax.broadcasted_iota(jnp.int32, (e, e), 0)
    ee_c = jax.lax.broadcasted_iota(jnp.int32, (e, e), 1)
    tri = (ee_r >= ee_c).astype(jnp.float32)
    pos = jax.lax.dot_general(
        tri, sel.astype(jnp.float32), (((1,), (0,)), ((), ())),
        preferred_element_type=jnp.float32).astype(jnp.int32)
    col_j = jax.lax.broadcasted_iota(jnp.int32, (e, e), 1)
    hits = jnp.logical_and(sel, (pos - 1) == col_j)
    e_row = jax.lax.broadcasted_iota(jnp.int32, (e, e), 0)
    sched = jnp.sum(jnp.where(hits, e_row, 0), axis=0, keepdims=True)
    n11 = pos[e - 1:e, :]
    e_col = jax.lax.broadcasted_iota(jnp.int32, (e, 1), 0)
    last = jnp.sum(jnp.where(jnp.logical_and(sel, pos == n11), e_col, 0),
                   axis=0, keepdims=True)
    j_row = jax.lax.broadcasted_iota(jnp.int32, (1, e), 1)
    sched_ref[...] = jnp.where(j_row < n11, sched, last)
    n_ref[...] = n11
    cw_ref[...] = jax.lax.dot_general(
        hits.astype(jnp.float32), comb, (((0,), (0,)), ((), ())),
        preferred_element_type=jnp.float32)


def _expert_kernel(sched_ref, n_ref, x_ref, w1_ref, wg_ref, w2_ref, cw_ref,
                   out_ref):
    del sched_ref
    i = pl.program_id(0)

    @pl.when(i == 0)
    def _():
        out_ref[...] = jnp.zeros_like(out_ref)

    @pl.when(i < n_ref[0])
    def _():
        x = x_ref[...]
        h1 = jnp.dot(x, w1_ref[0], preferred_element_type=jnp.float32)
        g = jnp.dot(x, wg_ref[0], preferred_element_type=jnp.float32)
        act = h1 * (g * jax.nn.sigmoid(g))
        oe = jnp.dot(act, w2_ref[0], preferred_element_type=jnp.float32)
        out_ref[...] += cw_ref[i] * oe


def kernel(x, router_w, w1, w_gate, w2):
    orig_shape = x.shape
    d = x.shape[-1]
    xf = x.reshape(-1, d)
    t = xf.shape[0]
    e = router_w.shape[0]
    h = w1.shape[2]

    sched, narr, cw = pl.pallas_call(
        _router_kernel,
        out_shape=(
            jax.ShapeDtypeStruct((1, e), jnp.int32),
            jax.ShapeDtypeStruct((1, 1), jnp.int32),
            jax.ShapeDtypeStruct((e, t), jnp.float32),
        ),
    )(xf, router_w)

    grid_spec = pltpu.PrefetchScalarGridSpec(
        num_scalar_prefetch=2,
        grid=(e,),
        in_specs=[
            pl.BlockSpec((t, d), lambda i, s, n: (0, 0)),
            pl.BlockSpec((1, d, h), lambda i, s, n: (s[0, i], 0, 0)),
            pl.BlockSpec((1, d, h), lambda i, s, n: (s[0, i], 0, 0)),
            pl.BlockSpec((1, h, d), lambda i, s, n: (s[0, i], 0, 0)),
            pl.BlockSpec((e, t, 1), lambda i, s, n: (0, 0, 0)),
        ],
        out_specs=pl.BlockSpec((t, d), lambda i, s, n: (0, 0)),
    )
    out = pl.pallas_call(
        _expert_kernel,
        grid_spec=grid_spec,
        out_shape=jax.ShapeDtypeStruct((t, d), jnp.float32),
    )(sched, narr.reshape((1,)), xf, w1, w_gate, w2,
      cw.reshape(e, t, 1))
    return out.reshape(orig_shape)


# speedup vs baseline: 1.2702x; 1.0170x over previous
import jax
import jax.numpy as jnp
from jax.experimental import pallas as pl
from jax.experimental.pallas import tpu as pltpu


def _router_kernel(x_ref, rw_ref, sched_ref, n_ref, cw_ref):
    x = x_ref[...]
    rw = rw_ref[...]
    logits = jax.lax.dot_general(
        rw, x, (((1,), (1,)), ((), ())), preferred_element_type=jnp.float32)
    e, t = logits.shape
    row = jax.lax.broadcasted_iota(jnp.int32, (e, t), 0)
    m1 = jnp.max(logits, axis=0, keepdims=True)
    i1 = jnp.min(jnp.where(logits == m1, row, e), axis=0, keepdims=True)
    masked = jnp.where(row == i1, -jnp.inf, logits)
    m2 = jnp.max(masked, axis=0, keepdims=True)
    i2 = jnp.min(jnp.where(masked == m2, row, e), axis=0, keepdims=True)
    b = jnp.exp(m2 - m1)
    w_hi = 1.0 / (1.0 + b)
    w_lo = b / (1.0 + b)
    comb = (jnp.where(row == i1, w_hi, 0.0)
            + jnp.where(row == i2, w_lo, 0.0))

    sel = jnp.max(comb, axis=1, keepdims=True) > 0.0
    ee_r = jax.lax.broadcasted_iota(jnp.int32, (e, e), 0)
    ee_c = jax.lax.broadcasted_iota(jnp.int32, (e, e), 1)
    tri = (ee_r >= ee_c).astype(jnp.float32)
    pos = jax.lax.dot_general(
        tri, sel.astype(jnp.float32), (((1,), (0,)), ((), ())),
        preferred_element_type=jnp.float32).astype(jnp.int32)
    col_j = jax.lax.broadcasted_iota(jnp.int32, (e, e), 1)
    hits = jnp.logical_and(sel, (pos - 1) == col_j)
    e_row = jax.lax.broadcasted_iota(jnp.int32, (e, e), 0)
    sched = jnp.sum(jnp.where(hits, e_row, 0), axis=0, keepdims=True)
    n11 = pos[e - 1:e, :]
    e_col = jax.lax.broadcasted_iota(jnp.int32, (e, 1), 0)
    last = jnp.sum(jnp.where(jnp.logical_and(sel, pos == n11), e_col, 0),
                   axis=0, keepdims=True)
    j_row = jax.lax.broadcasted_iota(jnp.int32, (1, e), 1)
    sched_ref[...] = jnp.where(j_row < n11, sched, last)
    n_ref[...] = n11
    cw_ref[...] = jax.lax.dot_general(
        hits.astype(jnp.float32), comb, (((0,), (0,)), ((), ())),
        preferred_element_type=jnp.float32)


def _expert_kernel(sched_ref, n_ref, x_ref, w1_ref, wg_ref, w2_ref, cw_ref,
                   out_ref):
    del sched_ref
    i = pl.program_id(0)

    @pl.when(i == 0)
    def _():
        out_ref[...] = jnp.zeros_like(out_ref)

    @pl.when(i < n_ref[0, 0])
    def _():
        x = x_ref[...]
        h1 = jnp.dot(x, w1_ref[0], preferred_element_type=jnp.float32)
        g = jnp.dot(x, wg_ref[0], preferred_element_type=jnp.float32)
        act = h1 * (g * jax.nn.sigmoid(g))
        oe = jnp.dot(act, w2_ref[0], preferred_element_type=jnp.float32)
        ccol = jnp.transpose(cw_ref[pl.ds(i, 1), :])
        out_ref[...] += ccol * oe


def kernel(x, router_w, w1, w_gate, w2):
    orig_shape = x.shape
    d = x.shape[-1]
    xf = x.reshape(-1, d)
    t = xf.shape[0]
    e = router_w.shape[0]
    h = w1.shape[2]

    sched, narr, cw = pl.pallas_call(
        _router_kernel,
        out_shape=(
            jax.ShapeDtypeStruct((1, e), jnp.int32),
            jax.ShapeDtypeStruct((1, 1), jnp.int32),
            jax.ShapeDtypeStruct((e, t), jnp.float32),
        ),
    )(xf, router_w)

    grid_spec = pltpu.PrefetchScalarGridSpec(
        num_scalar_prefetch=2,
        grid=(e,),
        in_specs=[
            pl.BlockSpec((t, d), lambda i, s, n: (0, 0)),
            pl.BlockSpec((1, d, h), lambda i, s, n: (s[0, i], 0, 0)),
            pl.BlockSpec((1, d, h), lambda i, s, n: (s[0, i], 0, 0)),
            pl.BlockSpec((1, h, d), lambda i, s, n: (s[0, i], 0, 0)),
            pl.BlockSpec((e, t), lambda i, s, n: (0, 0)),
        ],
        out_specs=pl.BlockSpec((t, d), lambda i, s, n: (0, 0)),
    )
    out = pl.pallas_call(
        _expert_kernel,
        grid_spec=grid_spec,
        out_shape=jax.ShapeDtypeStruct((t, d), jnp.float32),
    )(sched, narr, xf, w1, w_gate, w2, cw)
    return out.reshape(orig_shape)


# speedup vs baseline: 1.2848x; 1.0115x over previous
import jax
import jax.numpy as jnp
from jax.experimental import pallas as pl
from jax.experimental.pallas import tpu as pltpu


def _moe_kernel(x_ref, rw_ref, w1_hbm, wg_hbm, w2_hbm, out_ref,
                w1b, wgb, w2b, cwv, schedv, nv, sched_smem, n_smem,
                sems, ssem):
    x = x_ref[...]
    rw = rw_ref[...]
    logits = jax.lax.dot_general(
        rw, x, (((1,), (1,)), ((), ())), preferred_element_type=jnp.float32)
    e, t = logits.shape
    row = jax.lax.broadcasted_iota(jnp.int32, (e, t), 0)
    m1 = jnp.max(logits, axis=0, keepdims=True)
    i1 = jnp.min(jnp.where(logits == m1, row, e), axis=0, keepdims=True)
    masked = jnp.where(row == i1, -jnp.inf, logits)
    m2 = jnp.max(masked, axis=0, keepdims=True)
    i2 = jnp.min(jnp.where(masked == m2, row, e), axis=0, keepdims=True)
    b = jnp.exp(m2 - m1)
    w_hi = 1.0 / (1.0 + b)
    w_lo = b / (1.0 + b)
    comb = (jnp.where(row == i1, w_hi, 0.0)
            + jnp.where(row == i2, w_lo, 0.0))

    sel = jnp.max(comb, axis=1, keepdims=True) > 0.0
    ee_r = jax.lax.broadcasted_iota(jnp.int32, (e, e), 0)
    ee_c = jax.lax.broadcasted_iota(jnp.int32, (e, e), 1)
    tri = (ee_r >= ee_c).astype(jnp.float32)
    pos = jax.lax.dot_general(
        tri, sel.astype(jnp.float32), (((1,), (0,)), ((), ())),
        preferred_element_type=jnp.float32).astype(jnp.int32)
    hits = jnp.logical_and(sel, (pos - 1) == ee_c)
    sched = jnp.sum(jnp.where(hits, ee_r, 0), axis=0, keepdims=True)
    nv[...] = pos[e - 1:e, :]
    schedv[...] = sched
    cwv[...] = jax.lax.dot_general(
        hits.astype(jnp.float32), comb, (((0,), (0,)), ((), ())),
        preferred_element_type=jnp.float32)

    cps = pltpu.make_async_copy(schedv, sched_smem, ssem.at[0])
    cpn = pltpu.make_async_copy(nv, n_smem, ssem.at[1])
    cps.start()
    cpn.start()
    cps.wait()
    cpn.wait()
    n = n_smem[0, 0]

    def start_slot(j, slot):
        ej = sched_smem[0, j]
        pltpu.make_async_copy(w1_hbm.at[ej], w1b.at[slot], sems.at[slot, 0]).start()
        pltpu.make_async_copy(wg_hbm.at[ej], wgb.at[slot], sems.at[slot, 1]).start()
        pltpu.make_async_copy(w2_hbm.at[ej], w2b.at[slot], sems.at[slot, 2]).start()

    def wait_slot(j, slot):
        ej = sched_smem[0, j]
        pltpu.make_async_copy(w1_hbm.at[ej], w1b.at[slot], sems.at[slot, 0]).wait()
        pltpu.make_async_copy(wg_hbm.at[ej], wgb.at[slot], sems.at[slot, 1]).wait()
        pltpu.make_async_copy(w2_hbm.at[ej], w2b.at[slot], sems.at[slot, 2]).wait()

    start_slot(0, 0)
    out_ref[...] = jnp.zeros_like(out_ref)

    def body(j, carry):
        slot = jax.lax.rem(j, 2)

        @pl.when(j + 1 < n)
        def _():
            start_slot(j + 1, 1 - slot)

        wait_slot(j, slot)
        h1 = jnp.dot(x, w1b[slot], preferred_element_type=jnp.float32)
        g = jnp.dot(x, wgb[slot], preferred_element_type=jnp.float32)
        act = h1 * (g * jax.nn.sigmoid(g))
        oe = jnp.dot(act, w2b[slot], preferred_element_type=jnp.float32)
        ccol = jnp.transpose(cwv[pl.ds(j, 1), :])
        out_ref[...] += ccol * oe
        return carry

    jax.lax.fori_loop(0, n, body, 0)


def kernel(x, router_w, w1, w_gate, w2):
    orig_shape = x.shape
    d = x.shape[-1]
    xf = x.reshape(-1, d)
    t = xf.shape[0]
    e = router_w.shape[0]
    h = w1.shape[2]

    out = pl.pallas_call(
        _moe_kernel,
        in_specs=[
            pl.BlockSpec(memory_space=pltpu.VMEM),
            pl.BlockSpec(memory_space=pltpu.VMEM),
            pl.BlockSpec(memory_space=pltpu.HBM),
            pl.BlockSpec(memory_space=pltpu.HBM),
            pl.BlockSpec(memory_space=pltpu.HBM),
        ],
        out_specs=pl.BlockSpec(memory_space=pltpu.VMEM),
        out_shape=jax.ShapeDtypeStruct((t, d), jnp.float32),
        scratch_shapes=[
            pltpu.VMEM((2, d, h), jnp.float32),
            pltpu.VMEM((2, d, h), jnp.float32),
            pltpu.VMEM((2, h, d), jnp.float32),
            pltpu.VMEM((e, t), jnp.float32),
            pltpu.VMEM((1, e), jnp.int32),
            pltpu.VMEM((1, 1), jnp.int32),
            pltpu.SMEM((1, e), jnp.int32),
            pltpu.SMEM((1, 1), jnp.int32),
            pltpu.SemaphoreType.DMA((2, 3)),
            pltpu.SemaphoreType.DMA((2,)),
        ],
    )(xf, router_w, w1, w_gate, w2)
    return out.reshape(orig_shape)
